# baseline (speedup 1.0000x reference)
import functools

import jax
import jax.numpy as jnp
from jax import lax
from jax.experimental import pallas as pl
from jax.experimental.pallas import tpu as pltpu
from jax.experimental.pallas import tpu_sc as plsc

N = 10000
E = 320000
D = 128
NC = 2
NS = 16
NW = NC * NS
WIN = 80
NWIN = 128
CW = 16
NCHUNK = NWIN // CW
ZPAD = 240
PAD = NW * NWIN * WIN - E
SLAB = 624
TAIL_BASE = NS * SLAB
TAIL = N - TAIL_BASE


def _sc_partials(x_pad, src4, dst4, zeros):
    mesh = plsc.VectorSubcoreMesh(core_axis_name="c", subcore_axis_name="s")

    @functools.partial(
        pl.kernel,
        out_type=jax.ShapeDtypeStruct((NC, N, D), jnp.float32),
        mesh=mesh,
        scratch_types=[
            pltpu.VMEM((CW, WIN), jnp.int32),
            pltpu.VMEM((CW, WIN), jnp.int32),
            pltpu.VMEM((CW, WIN), jnp.int32),
            pltpu.VMEM((CW, WIN), jnp.int32),
            pltpu.VMEM((WIN, D), jnp.float32),
            pltpu.VMEM((WIN, D), jnp.float32),
            pltpu.VMEM((WIN, D), jnp.float32),
            pltpu.VMEM((WIN, D), jnp.float32),
            pltpu.VMEM_SHARED((N, D), jnp.float32),
            pltpu.SemaphoreType.DMA,
            pltpu.SemaphoreType.DMA,
            pltpu.SemaphoreType.DMA,
            pltpu.SemaphoreType.DMA,
            pltpu.SemaphoreType.DMA,
            pltpu.SemaphoreType.DMA,
            pltpu.SemaphoreType.DMA,
            pltpu.SemaphoreType.DMA,
            pltpu.SemaphoreType.DMA,
            pltpu.SemaphoreType.DMA,
        ],
    )
    def kern(x_hbm, src_hbm, dst_hbm, z_hbm, out_hbm,
             sc0, sc1, dc0, dc1, r0, r1, r2, r3, acc,
             g0, g1, g2, g3, s0, s1, s2, s3, i0, i1):
        src_c = (sc0, sc1)
        dst_c = (dc0, dc1)
        rows = (r0, r1, r2, r3)
        gsem = (g0, g1, g2, g3)
        ssem = (s0, s1, s2, s3)
        isem = (i0, i1)
        c = lax.axis_index("c")
        s = lax.axis_index("s")
        wid = c * NS + s

        pltpu.sync_copy(
            z_hbm.at[pl.ds(s * SLAB, SLAB)],
            acc.at[pl.ds(s * SLAB, SLAB)],
        )

        @pl.when(s == 0)
        def _():
            pltpu.sync_copy(
                z_hbm.at[pl.ds(TAIL_BASE, TAIL)],
                acc.at[pl.ds(TAIL_BASE, TAIL)],
            )
        plsc.subcore_barrier()

        pltpu.sync_copy(src_hbm.at[wid].at[0], src_c[0])
        pltpu.sync_copy(dst_hbm.at[wid].at[0], dst_c[0])
        for k in range(3):
            pltpu.async_copy(x_hbm.at[src_c[0].at[k]], rows[k], gsem[k])

        @pl.loop(0, NCHUNK, step=2)
        def _(ic):
            for u in range(2):
                ch = ic + u
                cb = u
                cbn = 1 - u
                for k in range(CW):
                    ww = ch * CW + k
                    b = k % 4
                    bp = (b - 1) % 4
                    bn = (b + 3) % 4
                    pltpu.make_async_copy(
                        x_hbm.at[src_c[cb].at[k]], rows[b], gsem[b]).wait()
                    pltpu.async_copy(rows[b], acc.at[dst_c[cb].at[k]],
                                     ssem[b], add=True)
                    if k == 0:
                        @pl.when(ww >= 1)
                        def _():
                            pltpu.make_async_copy(
                                rows[bp], acc.at[dst_c[cb].at[k]],
                                ssem[bp]).wait()

                        @pl.when(ch < NCHUNK - 1)
                        def _():
                            pltpu.async_copy(src_hbm.at[wid].at[ch + 1],
                                             src_c[cbn], isem[cbn])
                            pltpu.async_copy(dst_hbm.at[wid].at[ch + 1],
                                             dst_c[cbn], isem[cbn])
                    else:
                        pltpu.make_async_copy(
                            rows[bp], acc.at[dst_c[cb].at[k]],
                            ssem[bp]).wait()
                    if k == CW - 3:
                        @pl.when(ch < NCHUNK - 1)
                        def _():
                            pltpu.make_async_copy(
                                src_hbm.at[wid].at[0], src_c[cbn],
                                isem[cbn]).wait()
                            pltpu.make_async_copy(
                                dst_hbm.at[wid].at[0], dst_c[cbn],
                                isem[cbn]).wait()
                    if k < CW - 3:
                        pltpu.async_copy(
                            x_hbm.at[src_c[cb].at[k + 3]], rows[bn], gsem[bn])
                    else:
                        @pl.when(ch < NCHUNK - 1)
                        def _():
                            pltpu.async_copy(
                                x_hbm.at[src_c[cbn].at[k + 3 - CW]],
                                rows[bn], gsem[bn])

        pltpu.make_async_copy(rows[3], acc.at[dst_c[1].at[0]], ssem[3]).wait()
        plsc.subcore_barrier()

        pltpu.sync_copy(
            acc.at[pl.ds(s * SLAB, SLAB)],
            out_hbm.at[c].at[pl.ds(s * SLAB, SLAB)],
        )

        @pl.when(s == 0)
        def _():
            pltpu.sync_copy(
                acc.at[pl.ds(TAIL_BASE, TAIL)],
                out_hbm.at[c].at[pl.ds(TAIL_BASE, TAIL)],
            )

    return kern(x_pad, src4, dst4, zeros)


def _tc_mlp(x, partials, W1, b1, W2, b2, eps):
    BLK = 1000

    def body(x_ref, p_ref, w1_ref, b1_ref, w2_ref, b2_ref, eps_ref, o_ref):
        h = (1.0 + eps_ref[0, 0]) * x_ref[...] + p_ref[0] + p_ref[1]
        h = jnp.dot(h, w1_ref[...], preferred_element_type=jnp.float32)
        h = jnp.maximum(h + b1_ref[...], 0.0)
        h = jnp.dot(h, w2_ref[...], preferred_element_type=jnp.float32)
        o_ref[...] = h + b2_ref[...]

    return pl.pallas_call(
        body,
        grid=(N // BLK,),
        in_specs=[
            pl.BlockSpec((BLK, D), lambda i: (i, 0)),
            pl.BlockSpec((NC, BLK, D), lambda i: (0, i, 0)),
            pl.BlockSpec((D, D), lambda i: (0, 0)),
            pl.BlockSpec((1, D), lambda i: (0, 0)),
            pl.BlockSpec((D, D), lambda i: (0, 0)),
            pl.BlockSpec((1, D), lambda i: (0, 0)),
            pl.BlockSpec((1, 1), lambda i: (0, 0)),
        ],
        out_specs=pl.BlockSpec((BLK, D), lambda i: (i, 0)),
        out_shape=jax.ShapeDtypeStruct((N, D), jnp.float32),
    )(x, partials, W1, b1.reshape(1, D), W2, b2.reshape(1, D),
      eps.reshape(1, 1))


def kernel(x, edge_index, W1, b1, W2, b2, eps):
    src = edge_index[0].astype(jnp.int32)
    dst = edge_index[1].astype(jnp.int32)
    pad_iota = jnp.arange(PAD, dtype=jnp.int32)
    src4 = jnp.concatenate([src, N + pad_iota % ZPAD]).reshape(
        NW, NCHUNK, CW, WIN)
    dst4 = jnp.concatenate([dst, pad_iota % N]).reshape(NW, NCHUNK, CW, WIN)
    x_pad = jnp.concatenate([x, jnp.zeros((ZPAD, D), jnp.float32)], axis=0)
    zeros = jnp.zeros((N, D), jnp.float32)
    partials = _sc_partials(x_pad, src4, dst4, zeros)
    return _tc_mlp(x, partials, W1, b1, W2, b2, eps)


# speedup vs baseline: 1.0493x; 1.0493x over previous
import functools

import jax
import jax.numpy as jnp
from jax import lax
from jax.experimental import pallas as pl
from jax.experimental.pallas import tpu as pltpu
from jax.experimental.pallas import tpu_sc as plsc

N = 10000
E = 320000
D = 128
NC = 2
NS = 16
NW = NC * NS
WIN = 80
NWIN = 128
CW = 16
NCHUNK = NWIN // CW
ZDUM = 160
PAD = NW * NWIN * WIN - E
SLAB = 624
TAIL_BASE = NS * SLAB
TAIL = N - TAIL_BASE


def _sc_partials(x, src4, dst4, zeros):
    mesh = plsc.VectorSubcoreMesh(core_axis_name="c", subcore_axis_name="s")

    @functools.partial(
        pl.kernel,
        out_type=jax.ShapeDtypeStruct((NC, N, D), jnp.float32),
        mesh=mesh,
        scratch_types=[
            pltpu.VMEM((CW, WIN), jnp.int32),
            pltpu.VMEM((CW, WIN), jnp.int32),
            pltpu.VMEM((CW, WIN), jnp.int32),
            pltpu.VMEM((CW, WIN), jnp.int32),
            pltpu.VMEM((WIN, D), jnp.float32),
            pltpu.VMEM((WIN, D), jnp.float32),
            pltpu.VMEM((WIN, D), jnp.float32),
            pltpu.VMEM((WIN, D), jnp.float32),
            pltpu.VMEM_SHARED((N + ZDUM, D), jnp.float32),
            pltpu.SemaphoreType.DMA,
            pltpu.SemaphoreType.DMA,
            pltpu.SemaphoreType.DMA,
            pltpu.SemaphoreType.DMA,
            pltpu.SemaphoreType.DMA,
            pltpu.SemaphoreType.DMA,
            pltpu.SemaphoreType.DMA,
            pltpu.SemaphoreType.DMA,
            pltpu.SemaphoreType.DMA,
            pltpu.SemaphoreType.DMA,
        ],
    )
    def kern(x_hbm, src_hbm, dst_hbm, z_hbm, out_hbm,
             sc0, sc1, dc0, dc1, r0, r1, r2, r3, acc,
             g0, g1, g2, g3, s0, s1, s2, s3, i0, i1):
        src_c = (sc0, sc1)
        dst_c = (dc0, dc1)
        rows = (r0, r1, r2, r3)
        gsem = (g0, g1, g2, g3)
        ssem = (s0, s1, s2, s3)
        isem = (i0, i1)
        c = lax.axis_index("c")
        s = lax.axis_index("s")
        wid = c * NS + s

        pltpu.sync_copy(
            z_hbm.at[pl.ds(s * SLAB, SLAB)],
            acc.at[pl.ds(s * SLAB, SLAB)],
        )

        @pl.when(s == 0)
        def _():
            pltpu.sync_copy(
                z_hbm.at[pl.ds(TAIL_BASE, TAIL)],
                acc.at[pl.ds(TAIL_BASE, TAIL)],
            )
        plsc.subcore_barrier()

        pltpu.sync_copy(src_hbm.at[wid].at[0], src_c[0])
        pltpu.sync_copy(dst_hbm.at[wid].at[0], dst_c[0])
        for k in range(3):
            pltpu.async_copy(x_hbm.at[src_c[0].at[k]], rows[k], gsem[k])

        @pl.loop(0, NCHUNK, step=2)
        def _(ic):
            for u in range(2):
                ch = ic + u
                cb = u
                cbn = 1 - u
                for k in range(CW):
                    ww = ch * CW + k
                    b = k % 4
                    bp = (b - 1) % 4
                    bn = (b + 3) % 4
                    pltpu.make_async_copy(
                        x_hbm.at[src_c[cb].at[k]], rows[b], gsem[b]).wait()
                    pltpu.async_copy(rows[b], acc.at[dst_c[cb].at[k]],
                                     ssem[b], add=True)
                    if k == 0:
                        @pl.when(ww >= 1)
                        def _():
                            pltpu.make_async_copy(
                                rows[bp], acc.at[dst_c[cb].at[k]],
                                ssem[bp]).wait()

                        @pl.when(ch < NCHUNK - 1)
                        def _():
                            pltpu.async_copy(src_hbm.at[wid].at[ch + 1],
                                             src_c[cbn], isem[cbn])
                            pltpu.async_copy(dst_hbm.at[wid].at[ch + 1],
                                             dst_c[cbn], isem[cbn])
                    else:
                        pltpu.make_async_copy(
                            rows[bp], acc.at[dst_c[cb].at[k]],
                            ssem[bp]).wait()
                    if k == CW - 3:
                        @pl.when(ch < NCHUNK - 1)
                        def _():
                            pltpu.make_async_copy(
                                src_hbm.at[wid].at[0], src_c[cbn],
                                isem[cbn]).wait()
                            pltpu.make_async_copy(
                                dst_hbm.at[wid].at[0], dst_c[cbn],
                                isem[cbn]).wait()
                    if k < CW - 3:
                        pltpu.async_copy(
                            x_hbm.at[src_c[cb].at[k + 3]], rows[bn], gsem[bn])
                    else:
                        @pl.when(ch < NCHUNK - 1)
                        def _():
                            pltpu.async_copy(
                                x_hbm.at[src_c[cbn].at[k + 3 - CW]],
                                rows[bn], gsem[bn])

        pltpu.make_async_copy(rows[3], acc.at[dst_c[1].at[0]], ssem[3]).wait()
        plsc.subcore_barrier()

        pltpu.sync_copy(
            acc.at[pl.ds(s * SLAB, SLAB)],
            out_hbm.at[c].at[pl.ds(s * SLAB, SLAB)],
        )

        @pl.when(s == 0)
        def _():
            pltpu.sync_copy(
                acc.at[pl.ds(TAIL_BASE, TAIL)],
                out_hbm.at[c].at[pl.ds(TAIL_BASE, TAIL)],
            )

    return kern(x, src4, dst4, zeros)


def _tc_mlp(x, partials, W1, b1, W2, b2, eps):
    BLK = 2000

    def body(x_ref, p_ref, w1_ref, b1_ref, w2_ref, b2_ref, eps_ref, o_ref):
        h = (1.0 + eps_ref[0, 0]) * x_ref[...] + p_ref[0] + p_ref[1]
        h = jnp.dot(h, w1_ref[...], preferred_element_type=jnp.float32)
        h = jnp.maximum(h + b1_ref[...], 0.0)
        h = jnp.dot(h, w2_ref[...], preferred_element_type=jnp.float32)
        o_ref[...] = h + b2_ref[...]

    return pl.pallas_call(
        body,
        grid=(N // BLK,),
        in_specs=[
            pl.BlockSpec((BLK, D), lambda i: (i, 0)),
            pl.BlockSpec((NC, BLK, D), lambda i: (0, i, 0)),
            pl.BlockSpec((D, D), lambda i: (0, 0)),
            pl.BlockSpec((1, D), lambda i: (0, 0)),
            pl.BlockSpec((D, D), lambda i: (0, 0)),
            pl.BlockSpec((1, D), lambda i: (0, 0)),
            pl.BlockSpec((1, 1), lambda i: (0, 0)),
        ],
        out_specs=pl.BlockSpec((BLK, D), lambda i: (i, 0)),
        out_shape=jax.ShapeDtypeStruct((N, D), jnp.float32),
    )(x, partials, W1, b1.reshape(1, D), W2, b2.reshape(1, D),
      eps.reshape(1, 1))


def kernel(x, edge_index, W1, b1, W2, b2, eps):
    src = edge_index[0].astype(jnp.int32)
    dst = edge_index[1].astype(jnp.int32)
    pad_iota = jnp.arange(PAD, dtype=jnp.int32)
    src4 = jnp.concatenate([src, pad_iota % N]).reshape(NW, NCHUNK, CW, WIN)
    dst4 = jnp.concatenate([dst, N + pad_iota % ZDUM]).reshape(
        NW, NCHUNK, CW, WIN)
    zeros = jnp.zeros((N, D), jnp.float32)
    partials = _sc_partials(x, src4, dst4, zeros)
    return _tc_mlp(x, partials, W1, b1, W2, b2, eps)


# speedup vs baseline: 1.1062x; 1.0543x over previous
import functools

import jax
import jax.numpy as jnp
from jax import lax
from jax.experimental import pallas as pl
from jax.experimental.pallas import tpu as pltpu
from jax.experimental.pallas import tpu_sc as plsc

N = 10000
E = 320000
D = 128
NC = 2
NS = 16
NW = NC * NS
WIN = 80
NWIN = 128
CW = 16
NCHUNK = NWIN // CW
ZDUM = 160
PAD = NW * NWIN * WIN - E
SLAB = 624
TAIL_BASE = NS * SLAB
TAIL = N - TAIL_BASE


def _sc_partials(x, ei5, zeros):
    mesh = plsc.VectorSubcoreMesh(core_axis_name="c", subcore_axis_name="s")

    @functools.partial(
        pl.kernel,
        out_type=jax.ShapeDtypeStruct((NC, N, D), jnp.float32),
        mesh=mesh,
        scratch_types=[
            pltpu.VMEM((CW, WIN), jnp.int32),
            pltpu.VMEM((CW, WIN), jnp.int32),
            pltpu.VMEM((CW, WIN), jnp.int32),
            pltpu.VMEM((CW, WIN), jnp.int32),
            pltpu.VMEM((WIN, D), jnp.float32),
            pltpu.VMEM((WIN, D), jnp.float32),
            pltpu.VMEM((WIN, D), jnp.float32),
            pltpu.VMEM((WIN, D), jnp.float32),
            pltpu.VMEM_SHARED((N + ZDUM, D), jnp.float32),
            pltpu.SemaphoreType.DMA,
            pltpu.SemaphoreType.DMA,
            pltpu.SemaphoreType.DMA,
            pltpu.SemaphoreType.DMA,
            pltpu.SemaphoreType.DMA,
            pltpu.SemaphoreType.DMA,
            pltpu.SemaphoreType.DMA,
            pltpu.SemaphoreType.DMA,
            pltpu.SemaphoreType.DMA,
            pltpu.SemaphoreType.DMA,
        ],
    )
    def kern(x_hbm, ei_hbm, z_hbm, out_hbm,
             sc0, sc1, dc0, dc1, r0, r1, r2, r3, acc,
             g0, g1, g2, g3, s0, s1, s2, s3, i0, i1):
        src_hbm = ei_hbm.at[0]
        dst_hbm = ei_hbm.at[1]
        src_c = (sc0, sc1)
        dst_c = (dc0, dc1)
        rows = (r0, r1, r2, r3)
        gsem = (g0, g1, g2, g3)
        ssem = (s0, s1, s2, s3)
        isem = (i0, i1)
        c = lax.axis_index("c")
        s = lax.axis_index("s")
        wid = c * NS + s

        pltpu.sync_copy(
            z_hbm.at[pl.ds(s * SLAB, SLAB)],
            acc.at[pl.ds(s * SLAB, SLAB)],
        )

        @pl.when(s == 0)
        def _():
            pltpu.sync_copy(
                z_hbm.at[pl.ds(TAIL_BASE, TAIL)],
                acc.at[pl.ds(TAIL_BASE, TAIL)],
            )
        plsc.subcore_barrier()

        pltpu.sync_copy(src_hbm.at[wid].at[0], src_c[0])
        pltpu.sync_copy(dst_hbm.at[wid].at[0], dst_c[0])
        for k in range(3):
            pltpu.async_copy(x_hbm.at[src_c[0].at[k]], rows[k], gsem[k])

        @pl.loop(0, NCHUNK, step=2)
        def _(ic):
            for u in range(2):
                ch = ic + u
                cb = u
                cbn = 1 - u
                for k in range(CW):
                    ww = ch * CW + k
                    b = k % 4
                    bp = (b - 1) % 4
                    bn = (b + 3) % 4
                    pltpu.make_async_copy(
                        x_hbm.at[src_c[cb].at[k]], rows[b], gsem[b]).wait()
                    pltpu.async_copy(rows[b], acc.at[dst_c[cb].at[k]],
                                     ssem[b], add=True)
                    if k == 0:
                        @pl.when(ww >= 1)
                        def _():
                            pltpu.make_async_copy(
                                rows[bp], acc.at[dst_c[cb].at[k]],
                                ssem[bp]).wait()

                        @pl.when(ch < NCHUNK - 1)
                        def _():
                            pltpu.async_copy(src_hbm.at[wid].at[ch + 1],
                                             src_c[cbn], isem[cbn])
                            pltpu.async_copy(dst_hbm.at[wid].at[ch + 1],
                                             dst_c[cbn], isem[cbn])
                    else:
                        pltpu.make_async_copy(
                            rows[bp], acc.at[dst_c[cb].at[k]],
                            ssem[bp]).wait()
                    if k == CW - 3:
                        @pl.when(ch < NCHUNK - 1)
                        def _():
                            pltpu.make_async_copy(
                                src_hbm.at[wid].at[0], src_c[cbn],
                                isem[cbn]).wait()
                            pltpu.make_async_copy(
                                dst_hbm.at[wid].at[0], dst_c[cbn],
                                isem[cbn]).wait()
                    if k < CW - 3:
                        pltpu.async_copy(
                            x_hbm.at[src_c[cb].at[k + 3]], rows[bn], gsem[bn])
                    else:
                        @pl.when(ch < NCHUNK - 1)
                        def _():
                            pltpu.async_copy(
                                x_hbm.at[src_c[cbn].at[k + 3 - CW]],
                                rows[bn], gsem[bn])

        pltpu.make_async_copy(rows[3], acc.at[dst_c[1].at[0]], ssem[3]).wait()
        plsc.subcore_barrier()

        pltpu.sync_copy(
            acc.at[pl.ds(s * SLAB, SLAB)],
            out_hbm.at[c].at[pl.ds(s * SLAB, SLAB)],
        )

        @pl.when(s == 0)
        def _():
            pltpu.sync_copy(
                acc.at[pl.ds(TAIL_BASE, TAIL)],
                out_hbm.at[c].at[pl.ds(TAIL_BASE, TAIL)],
            )

    return kern(x, ei5, zeros)


def _tc_mlp(x, partials, W1, b1, W2, b2, eps):
    BLK = 2000

    def body(x_ref, p_ref, w1_ref, b1_ref, w2_ref, b2_ref, eps_ref, o_ref):
        h = (1.0 + eps_ref[0, 0]) * x_ref[...] + p_ref[0] + p_ref[1]
        h = jnp.dot(h, w1_ref[...], preferred_element_type=jnp.float32)
        h = jnp.maximum(h + b1_ref[...], 0.0)
        h = jnp.dot(h, w2_ref[...], preferred_element_type=jnp.float32)
        o_ref[...] = h + b2_ref[...]

    return pl.pallas_call(
        body,
        grid=(N // BLK,),
        in_specs=[
            pl.BlockSpec((BLK, D), lambda i: (i, 0)),
            pl.BlockSpec((NC, BLK, D), lambda i: (0, i, 0)),
            pl.BlockSpec((D, D), lambda i: (0, 0)),
            pl.BlockSpec((1, D), lambda i: (0, 0)),
            pl.BlockSpec((D, D), lambda i: (0, 0)),
            pl.BlockSpec((1, D), lambda i: (0, 0)),
            pl.BlockSpec((1, 1), lambda i: (0, 0)),
        ],
        out_specs=pl.BlockSpec((BLK, D), lambda i: (i, 0)),
        out_shape=jax.ShapeDtypeStruct((N, D), jnp.float32),
    )(x, partials, W1, b1.reshape(1, D), W2, b2.reshape(1, D),
      eps.reshape(1, 1))


def kernel(x, edge_index, W1, b1, W2, b2, eps):
    pad_iota = jnp.arange(PAD, dtype=jnp.int32)
    pad_pair = jnp.stack([pad_iota % N, N + pad_iota % ZDUM])
    ei5 = jnp.concatenate(
        [edge_index.astype(jnp.int32), pad_pair], axis=1).reshape(
        2, NW, NCHUNK, CW, WIN)
    zeros = jnp.zeros((N, D), jnp.float32)
    partials = _sc_partials(x, ei5, zeros)
    return _tc_mlp(x, partials, W1, b1, W2, b2, eps)


# speedup vs baseline: 1.1380x; 1.0287x over previous
import functools

import jax
import jax.numpy as jnp
from jax import lax
from jax.experimental import pallas as pl
from jax.experimental.pallas import tpu as pltpu
from jax.experimental.pallas import tpu_sc as plsc

N = 10000
E = 320000
D = 128
NC = 2
NS = 16
NW = NC * NS
WIN = 64
NWIN = 160
CW = 10
NCHUNK = NWIN // CW
NB = 5
ZDUM = 160
PAD = NW * NWIN * WIN - E
SLAB = 624
TAIL_BASE = NS * SLAB
TAIL = N - TAIL_BASE


def _sc_partials(x, ei5, zeros):
    mesh = plsc.VectorSubcoreMesh(core_axis_name="c", subcore_axis_name="s")

    @functools.partial(
        pl.kernel,
        out_type=jax.ShapeDtypeStruct((NC, N, D), jnp.float32),
        mesh=mesh,
        scratch_types=[
            pltpu.VMEM((CW, WIN), jnp.int32),
            pltpu.VMEM((CW, WIN), jnp.int32),
            pltpu.VMEM((CW, WIN), jnp.int32),
            pltpu.VMEM((CW, WIN), jnp.int32),
            pltpu.VMEM((WIN, D), jnp.float32),
            pltpu.VMEM((WIN, D), jnp.float32),
            pltpu.VMEM((WIN, D), jnp.float32),
            pltpu.VMEM((WIN, D), jnp.float32),
            pltpu.VMEM((WIN, D), jnp.float32),
            pltpu.VMEM_SHARED((N + ZDUM, D), jnp.float32),
            pltpu.SemaphoreType.DMA,
            pltpu.SemaphoreType.DMA,
            pltpu.SemaphoreType.DMA,
            pltpu.SemaphoreType.DMA,
            pltpu.SemaphoreType.DMA,
            pltpu.SemaphoreType.DMA,
            pltpu.SemaphoreType.DMA,
            pltpu.SemaphoreType.DMA,
            pltpu.SemaphoreType.DMA,
            pltpu.SemaphoreType.DMA,
            pltpu.SemaphoreType.DMA,
            pltpu.SemaphoreType.DMA,
        ],
    )
    def kern(x_hbm, ei_hbm, z_hbm, out_hbm,
             sc0, sc1, dc0, dc1, r0, r1, r2, r3, r4, acc,
             g0, g1, g2, g3, g4, s0, s1, s2, s3, s4, i0, i1):
        src_hbm = ei_hbm.at[0]
        dst_hbm = ei_hbm.at[1]
        src_c = (sc0, sc1)
        dst_c = (dc0, dc1)
        rows = (r0, r1, r2, r3, r4)
        gsem = (g0, g1, g2, g3, g4)
        ssem = (s0, s1, s2, s3, s4)
        isem = (i0, i1)
        c = lax.axis_index("c")
        s = lax.axis_index("s")
        wid = c * NS + s

        pltpu.sync_copy(
            z_hbm.at[pl.ds(s * SLAB, SLAB)],
            acc.at[pl.ds(s * SLAB, SLAB)],
        )

        @pl.when(s == 0)
        def _():
            pltpu.sync_copy(
                z_hbm.at[pl.ds(TAIL_BASE, TAIL)],
                acc.at[pl.ds(TAIL_BASE, TAIL)],
            )
        plsc.subcore_barrier()

        pltpu.sync_copy(src_hbm.at[wid].at[0], src_c[0])
        pltpu.sync_copy(dst_hbm.at[wid].at[0], dst_c[0])
        for k in range(NB - 1):
            pltpu.async_copy(x_hbm.at[src_c[0].at[k]], rows[k], gsem[k])

        @pl.loop(0, NCHUNK, step=2)
        def _(ic):
            for u in range(2):
                ch = ic + u
                cb = u
                cbn = 1 - u
                for k in range(CW):
                    ww = ch * CW + k
                    b = k % NB
                    bp = (b - 1) % NB
                    bn = (b + NB - 1) % NB
                    pltpu.make_async_copy(
                        x_hbm.at[src_c[cb].at[k]], rows[b], gsem[b]).wait()
                    pltpu.async_copy(rows[b], acc.at[dst_c[cb].at[k]],
                                     ssem[b], add=True)
                    if k == 0:
                        @pl.when(ww >= 1)
                        def _():
                            pltpu.make_async_copy(
                                rows[bp], acc.at[dst_c[cb].at[k]],
                                ssem[bp]).wait()

                        @pl.when(ch < NCHUNK - 1)
                        def _():
                            pltpu.async_copy(src_hbm.at[wid].at[ch + 1],
                                             src_c[cbn], isem[cbn])
                            pltpu.async_copy(dst_hbm.at[wid].at[ch + 1],
                                             dst_c[cbn], isem[cbn])
                    else:
                        pltpu.make_async_copy(
                            rows[bp], acc.at[dst_c[cb].at[k]],
                            ssem[bp]).wait()
                    if k == CW - (NB - 1):
                        @pl.when(ch < NCHUNK - 1)
                        def _():
                            pltpu.make_async_copy(
                                src_hbm.at[wid].at[0], src_c[cbn],
                                isem[cbn]).wait()
                            pltpu.make_async_copy(
                                dst_hbm.at[wid].at[0], dst_c[cbn],
                                isem[cbn]).wait()
                    if k < CW - (NB - 1):
                        pltpu.async_copy(
                            x_hbm.at[src_c[cb].at[k + NB - 1]],
                            rows[bn], gsem[bn])
                    else:
                        @pl.when(ch < NCHUNK - 1)
                        def _():
                            pltpu.async_copy(
                                x_hbm.at[src_c[cbn].at[k + NB - 1 - CW]],
                                rows[bn], gsem[bn])

        pltpu.make_async_copy(
            rows[(NWIN - 1) % NB], acc.at[dst_c[1].at[0]],
            ssem[(NWIN - 1) % NB]).wait()
        plsc.subcore_barrier()

        pltpu.sync_copy(
            acc.at[pl.ds(s * SLAB, SLAB)],
            out_hbm.at[c].at[pl.ds(s * SLAB, SLAB)],
        )

        @pl.when(s == 0)
        def _():
            pltpu.sync_copy(
                acc.at[pl.ds(TAIL_BASE, TAIL)],
                out_hbm.at[c].at[pl.ds(TAIL_BASE, TAIL)],
            )

    return kern(x, ei5, zeros)


def _tc_mlp(x, partials, W1, b1, W2, b2, eps):
    BLK = 2000

    def body(x_ref, p_ref, w1_ref, b1_ref, w2_ref, b2_ref, eps_ref, o_ref):
        h = (1.0 + eps_ref[0, 0]) * x_ref[...] + p_ref[0] + p_ref[1]
        h = jnp.dot(h, w1_ref[...], preferred_element_type=jnp.float32)
        h = jnp.maximum(h + b1_ref[...], 0.0)
        h = jnp.dot(h, w2_ref[...], preferred_element_type=jnp.float32)
        o_ref[...] = h + b2_ref[...]

    return pl.pallas_call(
        body,
        grid=(N // BLK,),
        in_specs=[
            pl.BlockSpec((BLK, D), lambda i: (i, 0)),
            pl.BlockSpec((NC, BLK, D), lambda i: (0, i, 0)),
            pl.BlockSpec((D, D), lambda i: (0, 0)),
            pl.BlockSpec((1, D), lambda i: (0, 0)),
            pl.BlockSpec((D, D), lambda i: (0, 0)),
            pl.BlockSpec((1, D), lambda i: (0, 0)),
            pl.BlockSpec((1, 1), lambda i: (0, 0)),
        ],
        out_specs=pl.BlockSpec((BLK, D), lambda i: (i, 0)),
        out_shape=jax.ShapeDtypeStruct((N, D), jnp.float32),
    )(x, partials, W1, b1.reshape(1, D), W2, b2.reshape(1, D),
      eps.reshape(1, 1))


def kernel(x, edge_index, W1, b1, W2, b2, eps):
    pad_iota = jnp.arange(PAD, dtype=jnp.int32)
    pad_pair = jnp.stack([pad_iota % N, N + pad_iota % ZDUM])
    ei5 = jnp.concatenate(
        [edge_index.astype(jnp.int32), pad_pair], axis=1).reshape(
        2, NW, NCHUNK, CW, WIN)
    zeros = jnp.zeros((N, D), jnp.float32)
    partials = _sc_partials(x, ei5, zeros)
    return _tc_mlp(x, partials, W1, b1, W2, b2, eps)


# speedup vs baseline: 1.1391x; 1.0009x over previous
import functools

import jax
import jax.numpy as jnp
from jax import lax
from jax.experimental import pallas as pl
from jax.experimental.pallas import tpu as pltpu
from jax.experimental.pallas import tpu_sc as plsc

N = 10000
E = 320000
D = 128
NC = 2
NS = 16
NW = NC * NS
WIN = 64
NWIN = 160
CW = 10
NCHUNK = NWIN // CW
NB = 5
ZDUM = 128
PAD = NW * NWIN * WIN - E
SLAB = 624
TAIL_BASE = NS * SLAB
TAIL = N - TAIL_BASE


def _sc_partials(x, ei5, zeros):
    mesh = plsc.VectorSubcoreMesh(core_axis_name="c", subcore_axis_name="s")

    @functools.partial(
        pl.kernel,
        out_type=jax.ShapeDtypeStruct((NC, N, D), jnp.float32),
        mesh=mesh,
        scratch_types=[
            pltpu.VMEM((CW, WIN), jnp.int32),
            pltpu.VMEM((CW, WIN), jnp.int32),
            pltpu.VMEM((CW, WIN), jnp.int32),
            pltpu.VMEM((CW, WIN), jnp.int32),
            pltpu.VMEM((WIN, D), jnp.float32),
            pltpu.VMEM((WIN, D), jnp.float32),
            pltpu.VMEM((WIN, D), jnp.float32),
            pltpu.VMEM((WIN, D), jnp.float32),
            pltpu.VMEM((WIN, D), jnp.float32),
            pltpu.VMEM_SHARED((N + ZDUM, D), jnp.float32),
            pltpu.SemaphoreType.DMA,
            pltpu.SemaphoreType.DMA,
            pltpu.SemaphoreType.DMA,
            pltpu.SemaphoreType.DMA,
            pltpu.SemaphoreType.DMA,
            pltpu.SemaphoreType.DMA,
            pltpu.SemaphoreType.DMA,
            pltpu.SemaphoreType.DMA,
            pltpu.SemaphoreType.DMA,
            pltpu.SemaphoreType.DMA,
            pltpu.SemaphoreType.DMA,
            pltpu.SemaphoreType.DMA,
        ],
    )
    def kern(x_hbm, ei_hbm, z_hbm, out_hbm,
             sc0, sc1, dc0, dc1, r0, r1, r2, r3, r4, acc,
             g0, g1, g2, g3, g4, s0, s1, s2, s3, s4, i0, i1):
        src_hbm = ei_hbm.at[0]
        dst_hbm = ei_hbm.at[1]
        src_c = (sc0, sc1)
        dst_c = (dc0, dc1)
        rows = (r0, r1, r2, r3, r4)
        gsem = (g0, g1, g2, g3, g4)
        ssem = (s0, s1, s2, s3, s4)
        isem = (i0, i1)
        c = lax.axis_index("c")
        s = lax.axis_index("s")
        wid = c * NS + s

        pltpu.sync_copy(
            z_hbm.at[pl.ds(s * SLAB, SLAB)],
            acc.at[pl.ds(s * SLAB, SLAB)],
        )

        @pl.when(s == 0)
        def _():
            pltpu.sync_copy(
                z_hbm.at[pl.ds(TAIL_BASE, TAIL)],
                acc.at[pl.ds(TAIL_BASE, TAIL)],
            )
        plsc.subcore_barrier()

        pltpu.sync_copy(src_hbm.at[wid].at[0], src_c[0])
        pltpu.sync_copy(dst_hbm.at[wid].at[0], dst_c[0])
        for k in range(NB - 1):
            pltpu.async_copy(x_hbm.at[src_c[0].at[k]], rows[k], gsem[k])

        @pl.loop(0, NCHUNK, step=2)
        def _(ic):
            for u in range(2):
                ch = ic + u
                cb = u
                cbn = 1 - u
                for k in range(CW):
                    ww = ch * CW + k
                    b = k % NB
                    bp = (b - 1) % NB
                    bn = (b + NB - 1) % NB
                    pltpu.make_async_copy(
                        x_hbm.at[src_c[cb].at[k]], rows[b], gsem[b]).wait()
                    pltpu.async_copy(rows[b], acc.at[dst_c[cb].at[k]],
                                     ssem[b], add=True)
                    if k == 0:
                        @pl.when(ww >= 1)
                        def _():
                            pltpu.make_async_copy(
                                rows[bp], acc.at[dst_c[cb].at[k]],
                                ssem[bp]).wait()

                        @pl.when(ch < NCHUNK - 1)
                        def _():
                            pltpu.async_copy(src_hbm.at[wid].at[ch + 1],
                                             src_c[cbn], isem[cbn])
                            pltpu.async_copy(dst_hbm.at[wid].at[ch + 1],
                                             dst_c[cbn], isem[cbn])
                    else:
                        pltpu.make_async_copy(
                            rows[bp], acc.at[dst_c[cb].at[k]],
                            ssem[bp]).wait()
                    if k == CW - (NB - 1):
                        @pl.when(ch < NCHUNK - 1)
                        def _():
                            pltpu.make_async_copy(
                                src_hbm.at[wid].at[0], src_c[cbn],
                                isem[cbn]).wait()
                            pltpu.make_async_copy(
                                dst_hbm.at[wid].at[0], dst_c[cbn],
                                isem[cbn]).wait()
                    if k < CW - (NB - 1):
                        pltpu.async_copy(
                            x_hbm.at[src_c[cb].at[k + NB - 1]],
                            rows[bn], gsem[bn])
                    else:
                        @pl.when(ch < NCHUNK - 1)
                        def _():
                            pltpu.async_copy(
                                x_hbm.at[src_c[cbn].at[k + NB - 1 - CW]],
                                rows[bn], gsem[bn])

        pltpu.make_async_copy(
            rows[(NWIN - 1) % NB], acc.at[dst_c[1].at[0]],
            ssem[(NWIN - 1) % NB]).wait()
        plsc.subcore_barrier()

        pltpu.sync_copy(
            acc.at[pl.ds(s * SLAB, SLAB)],
            out_hbm.at[c].at[pl.ds(s * SLAB, SLAB)],
        )

        @pl.when(s == 0)
        def _():
            pltpu.sync_copy(
                acc.at[pl.ds(TAIL_BASE, TAIL)],
                out_hbm.at[c].at[pl.ds(TAIL_BASE, TAIL)],
            )

    return kern(x, ei5, zeros)


def _tc_mlp(x, partials, W1, b1, W2, b2, eps):
    BLK = 2000

    def body(x_ref, p_ref, w1_ref, b1_ref, w2_ref, b2_ref, eps_ref, o_ref):
        h = (1.0 + eps_ref[0, 0]) * x_ref[...] + p_ref[0] + p_ref[1]
        h = jnp.dot(h, w1_ref[...], preferred_element_type=jnp.float32)
        h = jnp.maximum(h + b1_ref[...], 0.0)
        h = jnp.dot(h, w2_ref[...], preferred_element_type=jnp.float32)
        o_ref[...] = h + b2_ref[...]

    return pl.pallas_call(
        body,
        grid=(N // BLK,),
        in_specs=[
            pl.BlockSpec((BLK, D), lambda i: (i, 0)),
            pl.BlockSpec((NC, BLK, D), lambda i: (0, i, 0)),
            pl.BlockSpec((D, D), lambda i: (0, 0)),
            pl.BlockSpec((1, D), lambda i: (0, 0)),
            pl.BlockSpec((D, D), lambda i: (0, 0)),
            pl.BlockSpec((1, D), lambda i: (0, 0)),
            pl.BlockSpec((1, 1), lambda i: (0, 0)),
        ],
        out_specs=pl.BlockSpec((BLK, D), lambda i: (i, 0)),
        out_shape=jax.ShapeDtypeStruct((N, D), jnp.float32),
    )(x, partials, W1, b1.reshape(1, D), W2, b2.reshape(1, D),
      eps.reshape(1, 1))


def kernel(x, edge_index, W1, b1, W2, b2, eps):
    pad_iota = jnp.arange(PAD, dtype=jnp.int32)
    pad_pair = jnp.stack([pad_iota, N + (pad_iota & (ZDUM - 1))])
    ei5 = jnp.concatenate(
        [edge_index.astype(jnp.int32), pad_pair], axis=1).reshape(
        2, NW, NCHUNK, CW, WIN)
    zeros = jnp.zeros((N, D), jnp.float32)
    partials = _sc_partials(x, ei5, zeros)
    return _tc_mlp(x, partials, W1, b1, W2, b2, eps)
